# baseline (speedup 1.0000x reference)
import jax
import jax.numpy as jnp
from jax import lax
from jax.experimental import pallas as pl
from jax.experimental.pallas import tpu as pltpu
from jax.experimental.pallas import tpu_sc as plsc

N = 10000
E = 320000
D = 128

NC, NS = 2, 16
NW = NC * NS
R = 320
NP = NW * R
NB = 336

CHUNK = 8000
KCH = E // CHUNK
FB = 8192
SEG = FB
OB = FB + CHUNK + 32
CAP = E + 2 * FB
MAXSEG = (E + SEG - 1) // SEG

C = 128
SEGC = SEG // C


def _mesh():
    return plsc.VectorSubcoreMesh(core_axis_name="c", subcore_axis_name="s")


def _wid():
    return lax.axis_index("s") * NC + lax.axis_index("c")


def _bin_body(src_hbm, dst_hbm, ls_hbm, ld_hbm, cnt_hbm, cnt2_hbm,
              sbuf, dbuf, obuf_s, obuf_d, hist, pos, tmpa, tmpb, cvec, sems):
    w = _wid()
    lo = w * R
    nv = CHUNK // 16
    lane = lax.broadcasted_iota(jnp.int32, (16,), 0)

    def in_copies(k):
        slot = k % 2
        return (
            pltpu.make_async_copy(
                src_hbm.at[pl.ds(pl.multiple_of(k * CHUNK, 8), CHUNK)],
                sbuf.at[pl.ds(slot * CHUNK, CHUNK)], sems.at[slot, 0]),
            pltpu.make_async_copy(
                dst_hbm.at[pl.ds(pl.multiple_of(k * CHUNK, 8), CHUNK)],
                dbuf.at[pl.ds(slot * CHUNK, CHUNK)], sems.at[slot, 1]),
        )

    for cp in in_copies(0):
        cp.start()
    p_vec = jnp.zeros((16,), jnp.int32)
    fl = jnp.int32(0)

    for k in range(KCH):
        slot = k % 2
        if k + 1 < KCH:
            for cp in in_copies(k + 1):
                cp.start()
        for cp in in_copies(k):
            cp.wait()

        def vec_body(v, p_vec):
            s = sbuf[pl.ds(slot * CHUNK + v * 16, 16)]
            d = dbuf[pl.ds(slot * CHUNK + v * 16, 16)]
            m = (d >= lo) & (d < lo + R)
            mi = m.astype(jnp.int32)
            excl = plsc.cumsum(mi) - mi
            addr = excl + p_vec
            plsc.store_scatter(obuf_s, [addr], s, mask=m)
            plsc.store_scatter(obuf_d, [addr], d - lo, mask=m)
            pc = plsc.all_reduce_population_count(m)
            return p_vec + pc

        p_vec = lax.fori_loop(0, nv, vec_body, p_vec)

        p_s = jnp.max(p_vec)

        def do_flush(args):
            p_vec, fl = args
            pltpu.sync_copy(obuf_s.at[pl.ds(0, FB)],
                            ls_hbm.at[pl.ds(pl.multiple_of(w * CAP + fl, 8), FB)])
            pltpu.sync_copy(obuf_d.at[pl.ds(0, FB)],
                            ld_hbm.at[pl.ds(pl.multiple_of(w * CAP + fl, 8), FB)])
            nshift = (jnp.max(p_vec) - FB + 15) // 16

            def shift(j, _):
                obuf_s[pl.ds(j * 16, 16)] = obuf_s[pl.ds(FB + j * 16, 16)]
                obuf_d[pl.ds(j * 16, 16)] = obuf_d[pl.ds(FB + j * 16, 16)]
                return 0

            lax.fori_loop(0, nshift, shift, 0)
            return p_vec - FB, fl + FB

        p_vec, fl = lax.cond(p_s >= FB, do_flush, lambda a: a, (p_vec, fl))

    p_s = jnp.max(p_vec)

    def sent(j, _):
        g = j * 16 + lane
        vs = obuf_s[pl.ds(j * 16, 16)]
        vd = obuf_d[pl.ds(j * 16, 16)]
        obuf_s[pl.ds(j * 16, 16)] = jnp.where(g >= p_s, 0, vs)
        obuf_d[pl.ds(j * 16, 16)] = jnp.where(g >= p_s, R, vd)
        return 0

    lax.fori_loop(0, FB // 16, sent, 0)
    pltpu.sync_copy(obuf_s.at[pl.ds(0, FB)],
                    ls_hbm.at[pl.ds(pl.multiple_of(w * CAP + fl, 8), FB)])
    pltpu.sync_copy(obuf_d.at[pl.ds(0, FB)],
                    ld_hbm.at[pl.ds(pl.multiple_of(w * CAP + fl, 8), FB)])

    def sent2(j, _):
        obuf_s[pl.ds(j * 16, 16)] = jnp.zeros((16,), jnp.int32)
        obuf_d[pl.ds(j * 16, 16)] = jnp.full((16,), R, jnp.int32)
        return 0

    lax.fori_loop(0, FB // 16, sent2, 0)
    pltpu.sync_copy(obuf_s.at[pl.ds(0, FB)],
                    ls_hbm.at[pl.ds(pl.multiple_of(w * CAP + fl + FB, 8), FB)])
    pltpu.sync_copy(obuf_d.at[pl.ds(0, FB)],
                    ld_hbm.at[pl.ds(pl.multiple_of(w * CAP + fl + FB, 8), FB)])

    n = fl + p_s
    cvec[pl.ds(0, 16)] = jnp.broadcast_to(n, (16,))
    pltpu.sync_copy(cvec, cnt_hbm.at[pl.ds(pl.multiple_of(w * 16, 8), 16)])

    nseg = (n + SEG - 1) // SEG
    prevsel = jnp.maximum(lane - 1, 0)
    nextsel = jnp.minimum(lane + 1, 15)
    ones = jnp.ones((16,), jnp.int32)

    def rank_of(dl):
        sk, perm = plsc.sort_key_val(dl, lane)
        tmpa[pl.ds(0, 16)] = sk
        prev = plsc.load_gather(tmpa, [prevsel])
        newg = jnp.where(lane == 0, ones, (sk != prev).astype(jnp.int32))
        start = plsc.cummax(jnp.where(newg == 1, lane, 0))
        rank = lane - start
        tmpb[pl.ds(0, 16)] = newg
        nxt = plsc.load_gather(tmpb, [nextsel])
        islast = (lane == 15) | (nxt == 1)
        return sk, perm, rank, islast

    def seg_body(s, _):
        sb = pl.multiple_of(w * CAP + s * SEG, 8)
        pltpu.sync_copy(ls_hbm.at[pl.ds(sb, SEG)], obuf_s.at[pl.ds(0, SEG)])
        pltpu.sync_copy(ld_hbm.at[pl.ds(sb, SEG)], obuf_d.at[pl.ds(0, SEG)])

        for j in range(NB // 16):
            hist[pl.ds(j * 16, 16)] = jnp.zeros((16,), jnp.int32)

        def hvec(v, _):
            dl = obuf_d[pl.ds(v * 16, 16)]
            sk, _, rank, islast = rank_of(dl)
            plsc.addupdate_scatter(hist, [sk], rank + 1, mask=islast)
            return 0

        lax.fori_loop(0, SEG // 16, hvec, 0)

        pltpu.sync_copy(
            hist,
            cnt2_hbm.at[pl.ds(pl.multiple_of((w * MAXSEG + s) * NB, 8), NB)])

        run = jnp.int32(0)
        for j in range(NB // 16):
            h = hist[pl.ds(j * 16, 16)]
            cs = plsc.cumsum(h)
            pos[pl.ds(j * 16, 16)] = run + cs - h
            run = run + jnp.max(cs)

        def svec(v, _):
            dl = obuf_d[pl.ds(v * 16, 16)]
            sv = obuf_s[pl.ds(v * 16, 16)]
            sk, perm, rank, islast = rank_of(dl)
            base = plsc.load_gather(pos, [sk])
            mypos = base + rank
            tmpa[pl.ds(0, 16)] = sv
            svals = plsc.load_gather(tmpa, [perm])
            plsc.store_scatter(sbuf, [mypos], svals)
            plsc.store_scatter(pos, [sk], mypos + 1, mask=islast)
            return 0

        lax.fori_loop(0, SEG // 16, svec, 0)
        pltpu.sync_copy(sbuf.at[pl.ds(0, SEG)], ls_hbm.at[pl.ds(sb, SEG)])
        return 0

    lax.fori_loop(0, nseg, seg_body, 0)


@jax.jit
def _bin_edges(src, dst):
    f = pl.kernel(
        _bin_body,
        out_type=(
            jax.ShapeDtypeStruct((NW * CAP,), jnp.int32),
            jax.ShapeDtypeStruct((NW * CAP,), jnp.int32),
            jax.ShapeDtypeStruct((NW * 16,), jnp.int32),
            jax.ShapeDtypeStruct((NW * MAXSEG * NB,), jnp.int32),
        ),
        mesh=_mesh(),
        compiler_params=pltpu.CompilerParams(needs_layout_passes=False),
        scratch_types=[
            pltpu.VMEM((2 * CHUNK,), jnp.int32),
            pltpu.VMEM((2 * CHUNK,), jnp.int32),
            pltpu.VMEM((OB,), jnp.int32),
            pltpu.VMEM((OB,), jnp.int32),
            pltpu.VMEM((NB,), jnp.int32),
            pltpu.VMEM((NB,), jnp.int32),
            pltpu.VMEM((16,), jnp.int32),
            pltpu.VMEM((16,), jnp.int32),
            pltpu.VMEM((16,), jnp.int32),
            pltpu.SemaphoreType.DMA((2, 2)),
        ],
    )
    return f(src, dst)


def _segmax_body(hp_hbm, ls_hbm, cnt_hbm, cnt2_hbm, out_hbm,
                 agg, rows0, rows1, rows2, rows3,
                 six0, six1, six2, six3, cnts, cvec, gsem, isem):
    w = _wid()
    rows = [rows0, rows1, rows2, rows3]
    six = [six0, six1, six2, six3]

    ninf = jnp.full((16,), -jnp.inf, jnp.float32)

    def init(r, _):
        for c in range(D // 16):
            agg[r, pl.ds(c * 16, 16)] = ninf
        return 0

    lax.fori_loop(0, R + 1, init, 0)

    pltpu.sync_copy(cnt_hbm.at[pl.ds(pl.multiple_of(w * 16, 8), 16)], cvec)
    n = jnp.max(cvec[pl.ds(0, 16)])
    nch = (n + C - 1) // C
    last = jnp.maximum(nch - 1, 0)

    pltpu.sync_copy(
        cnt2_hbm.at[pl.ds(pl.multiple_of(w * MAXSEG * NB, 8), MAXSEG * NB)],
        cnts)

    def cc(k):
        return jnp.minimum(k, last)

    def idx_copy(ch, slot):
        base = pl.multiple_of(w * CAP + ch * C, 8)
        return pltpu.make_async_copy(ls_hbm.at[pl.ds(base, C)], six[slot], isem)

    def gather(slot):
        return pltpu.make_async_copy(hp_hbm.at[six[slot]], rows[slot], gsem)

    def cnt_at(idx):
        v = plsc.load_gather(
            cnts, [jnp.broadcast_to(idx, (16,)).astype(jnp.int32)])
        return jnp.max(v)

    def process(g, slot, state):
        rbuf = rows[slot]

        def do(state):
            r, rem = state
            sgb = (g // SEGC) * NB
            r, rem = lax.cond(
                g % SEGC == 0,
                lambda s: (jnp.int32(0), cnt_at(sgb)),
                lambda s: s, (r, rem))
            left = jnp.minimum(C, n - g * C)

            def outer_cond(st):
                return st[0] < left

            def outer(st):
                e, r, rem = st

                def adv_cond(st2):
                    r2, rem2 = st2
                    return (rem2 == 0) & (r2 < NB - 1)

                def adv(st2):
                    r2, _ = st2
                    return r2 + 1, cnt_at(sgb + r2 + 1)

                r, rem = lax.while_loop(adv_cond, adv, (r, rem))
                k = jnp.minimum(rem, left - e)
                acc = tuple(agg[r, pl.ds(c * 16, 16)] for c in range(D // 16))

                def red(i, acc):
                    return tuple(
                        jnp.maximum(acc[c], rbuf[e + i, pl.ds(c * 16, 16)])
                        for c in range(D // 16))

                acc = lax.fori_loop(0, k, red, acc)
                for c in range(D // 16):
                    agg[r, pl.ds(c * 16, 16)] = acc[c]
                return e + k, r, rem - k

            e, r, rem = lax.while_loop(outer_cond, outer,
                                       (jnp.int32(0), r, rem))
            return r, rem

        return lax.cond(g < nch, do, lambda s: s, state)

    for b in range(4):
        idx_copy(cc(b), b).start()
    for b in range(3):
        idx_copy(cc(b), b).wait()
        gather(b).start()

    ng4 = (nch + 3) // 4

    def quad(g4, state):
        gq = g4 * 4
        for b in range(4):
            g = gq + b
            idx_copy(cc(g + 3), (b + 3) % 4).wait()
            gather((b + 3) % 4).start()
            gather(b).wait()
            idx_copy(cc(g + 4), b).start()
            state = process(g, b, state)
        return state

    lax.fori_loop(0, jnp.maximum(ng4, 1), quad,
                  (jnp.int32(0), jnp.int32(0)))

    for b in range(3):
        gather(b).wait()
    idx_copy(cc(0), 0).wait()

    pltpu.sync_copy(agg.at[pl.ds(0, R)],
                    out_hbm.at[pl.ds(pl.multiple_of(w * R, 8), R)])


@jax.jit
def _segmax(hp, ls, cnt, cnt2):
    f = pl.kernel(
        _segmax_body,
        out_type=jax.ShapeDtypeStruct((NP, D), jnp.float32),
        mesh=_mesh(),
        compiler_params=pltpu.CompilerParams(needs_layout_passes=False),
        scratch_types=(
            [pltpu.VMEM((R + 1, D), jnp.float32)]
            + [pltpu.VMEM((C, D), jnp.float32) for _ in range(4)]
            + [pltpu.VMEM((C,), jnp.int32) for _ in range(4)]
            + [pltpu.VMEM((MAXSEG * NB,), jnp.int32),
               pltpu.VMEM((16,), jnp.int32),
               pltpu.SemaphoreType.DMA,
               pltpu.SemaphoreType.DMA]
        ),
    )
    return f(hp, ls, cnt, cnt2)[:N]


BM = 1000


def _l2norm(h):
    return h / jnp.maximum(
        jnp.sqrt(jnp.sum(h * h, axis=-1, keepdims=True)), 1e-12)


def _pool_body(h_ref, w_ref, b_ref, o_ref):
    o_ref[...] = jnp.maximum(
        jnp.dot(h_ref[...], w_ref[...], preferred_element_type=jnp.float32)
        + b_ref[...], 0.0)


@jax.jit
def _pool_mm(h, Wp, bp):
    return pl.pallas_call(
        _pool_body,
        grid=(N // BM,),
        in_specs=[
            pl.BlockSpec((BM, D), lambda i: (i, 0)),
            pl.BlockSpec((D, D), lambda i: (0, 0)),
            pl.BlockSpec((1, D), lambda i: (0, 0)),
        ],
        out_specs=pl.BlockSpec((BM, D), lambda i: (i, 0)),
        out_shape=jax.ShapeDtypeStruct((N, D), jnp.float32),
    )(h, Wp, bp.reshape(1, D))


def _combine_body(h_ref, a_ref, ws_ref, wn_ref, b_ref, wp_ref, bp_ref,
                  h1_ref, hp1_ref):
    a = a_ref[...]
    a = jnp.where(jnp.isfinite(a), a, 0.0)
    r = (jnp.dot(h_ref[...], ws_ref[...], preferred_element_type=jnp.float32)
         + jnp.dot(a, wn_ref[...], preferred_element_type=jnp.float32)
         + b_ref[...])
    h1 = _l2norm(jnp.maximum(r, 0.0))
    h1_ref[...] = h1
    hp1_ref[...] = jnp.maximum(
        jnp.dot(h1, wp_ref[...], preferred_element_type=jnp.float32)
        + bp_ref[...], 0.0)


@jax.jit
def _combine_pool(h, agg, Ws, Wn, b, Wp, bp):
    return pl.pallas_call(
        _combine_body,
        grid=(N // BM,),
        in_specs=[
            pl.BlockSpec((BM, D), lambda i: (i, 0)),
            pl.BlockSpec((BM, D), lambda i: (i, 0)),
            pl.BlockSpec((D, D), lambda i: (0, 0)),
            pl.BlockSpec((D, D), lambda i: (0, 0)),
            pl.BlockSpec((1, D), lambda i: (0, 0)),
            pl.BlockSpec((D, D), lambda i: (0, 0)),
            pl.BlockSpec((1, D), lambda i: (0, 0)),
        ],
        out_specs=[
            pl.BlockSpec((BM, D), lambda i: (i, 0)),
            pl.BlockSpec((BM, D), lambda i: (i, 0)),
        ],
        out_shape=[
            jax.ShapeDtypeStruct((N, D), jnp.float32),
            jax.ShapeDtypeStruct((N, D), jnp.float32),
        ],
    )(h, agg, Ws, Wn, b.reshape(1, D), Wp, bp.reshape(1, D))


def _final_body(h_ref, a_ref, ws_ref, wn_ref, b_ref, o_ref):
    a = a_ref[...]
    a = jnp.where(jnp.isfinite(a), a, 0.0)
    r = (jnp.dot(h_ref[...], ws_ref[...], preferred_element_type=jnp.float32)
         + jnp.dot(a, wn_ref[...], preferred_element_type=jnp.float32)
         + b_ref[...])
    m = jnp.max(r, axis=-1, keepdims=True)
    ls = r - m - jnp.log(jnp.sum(jnp.exp(r - m), axis=-1, keepdims=True))
    o_ref[...] = _l2norm(ls)


@jax.jit
def _final(h, agg, Ws, Wn, b):
    do = Ws.shape[1]
    return pl.pallas_call(
        _final_body,
        grid=(N // BM,),
        in_specs=[
            pl.BlockSpec((BM, D), lambda i: (i, 0)),
            pl.BlockSpec((BM, D), lambda i: (i, 0)),
            pl.BlockSpec((D, do), lambda i: (0, 0)),
            pl.BlockSpec((D, do), lambda i: (0, 0)),
            pl.BlockSpec((1, do), lambda i: (0, 0)),
        ],
        out_specs=pl.BlockSpec((BM, do), lambda i: (i, 0)),
        out_shape=jax.ShapeDtypeStruct((N, do), jnp.float32),
    )(h, agg, Ws, Wn, b.reshape(1, do))


def kernel(x, edge_index, Wp0, bp0, Wn0, Ws0, b0,
           Wp1, bp1, Wn1, Ws1, b1, Wp2, bp2, Wn2, Ws2, b2):
    src = edge_index[0]
    dst = edge_index[1]
    ls, ld, cnt, cnt2 = _bin_edges(src, dst)
    hp0 = _pool_mm(x, Wp0, bp0)
    agg0 = _segmax(hp0, ls, cnt, cnt2)
    h1, hp1 = _combine_pool(x, agg0, Ws0, Wn0, b0, Wp1, bp1)
    agg1 = _segmax(hp1, ls, cnt, cnt2)
    h2, hp2 = _combine_pool(h1, agg1, Ws1, Wn1, b1, Wp2, bp2)
    agg2 = _segmax(hp2, ls, cnt, cnt2)
    return _final(h2, agg2, Ws2, Wn2, b2)


# speedup vs baseline: 1.3345x; 1.3345x over previous
import jax
import jax.numpy as jnp
from jax import lax
from jax.experimental import pallas as pl
from jax.experimental.pallas import tpu as pltpu
from jax.experimental.pallas import tpu_sc as plsc

N = 10000
E = 320000
D = 128

NC, NS = 2, 16
NW = NC * NS
R = 320
NP = NW * R
NB = 336

CHUNK = 8000
KCH = E // CHUNK
FB = 8192
SEG = FB
OB = FB + CHUNK + 32
CAP = E + 2 * FB
MAXSEG = (E + SEG - 1) // SEG

C = 128
SEGC = SEG // C


def _mesh():
    return plsc.VectorSubcoreMesh(core_axis_name="c", subcore_axis_name="s")


def _wid():
    return lax.axis_index("s") * NC + lax.axis_index("c")


def _bin_body(src_hbm, dst_hbm, ls_hbm, ld_hbm, cnt_hbm, cnt2_hbm,
              sbuf, dbuf, obuf_s, obuf_d, hist, pos, tmpa, tmpb, cvec, sems):
    w = _wid()
    lo = w * R
    nv = CHUNK // 16
    lane = lax.broadcasted_iota(jnp.int32, (16,), 0)

    def in_copies(k):
        slot = k % 2
        return (
            pltpu.make_async_copy(
                src_hbm.at[pl.ds(pl.multiple_of(k * CHUNK, 8), CHUNK)],
                sbuf.at[pl.ds(slot * CHUNK, CHUNK)], sems.at[slot, 0]),
            pltpu.make_async_copy(
                dst_hbm.at[pl.ds(pl.multiple_of(k * CHUNK, 8), CHUNK)],
                dbuf.at[pl.ds(slot * CHUNK, CHUNK)], sems.at[slot, 1]),
        )

    for cp in in_copies(0):
        cp.start()
    p_vec = jnp.zeros((16,), jnp.int32)
    fl = jnp.int32(0)

    for k in range(KCH):
        slot = k % 2
        if k + 1 < KCH:
            for cp in in_copies(k + 1):
                cp.start()
        for cp in in_copies(k):
            cp.wait()

        def vec_body(v, p_vec):
            s = sbuf[pl.ds(slot * CHUNK + v * 16, 16)]
            d = dbuf[pl.ds(slot * CHUNK + v * 16, 16)]
            m = (d >= lo) & (d < lo + R)
            mi = m.astype(jnp.int32)
            excl = plsc.cumsum(mi) - mi
            addr = excl + p_vec
            plsc.store_scatter(obuf_s, [addr], s, mask=m)
            plsc.store_scatter(obuf_d, [addr], d - lo, mask=m)
            pc = plsc.all_reduce_population_count(m)
            return p_vec + pc

        p_vec = lax.fori_loop(0, nv, vec_body, p_vec)

        p_s = jnp.max(p_vec)

        def do_flush(args):
            p_vec, fl = args
            pltpu.sync_copy(obuf_s.at[pl.ds(0, FB)],
                            ls_hbm.at[pl.ds(pl.multiple_of(w * CAP + fl, 8), FB)])
            pltpu.sync_copy(obuf_d.at[pl.ds(0, FB)],
                            ld_hbm.at[pl.ds(pl.multiple_of(w * CAP + fl, 8), FB)])
            nshift = (jnp.max(p_vec) - FB + 15) // 16

            def shift(j, _):
                obuf_s[pl.ds(j * 16, 16)] = obuf_s[pl.ds(FB + j * 16, 16)]
                obuf_d[pl.ds(j * 16, 16)] = obuf_d[pl.ds(FB + j * 16, 16)]
                return 0

            lax.fori_loop(0, nshift, shift, 0)
            return p_vec - FB, fl + FB

        p_vec, fl = lax.cond(p_s >= FB, do_flush, lambda a: a, (p_vec, fl))

    p_s = jnp.max(p_vec)

    def sent(j, _):
        g = j * 16 + lane
        vs = obuf_s[pl.ds(j * 16, 16)]
        vd = obuf_d[pl.ds(j * 16, 16)]
        obuf_s[pl.ds(j * 16, 16)] = jnp.where(g >= p_s, 0, vs)
        obuf_d[pl.ds(j * 16, 16)] = jnp.where(g >= p_s, R, vd)
        return 0

    lax.fori_loop(0, FB // 16, sent, 0)
    pltpu.sync_copy(obuf_s.at[pl.ds(0, FB)],
                    ls_hbm.at[pl.ds(pl.multiple_of(w * CAP + fl, 8), FB)])
    pltpu.sync_copy(obuf_d.at[pl.ds(0, FB)],
                    ld_hbm.at[pl.ds(pl.multiple_of(w * CAP + fl, 8), FB)])

    def sent2(j, _):
        obuf_s[pl.ds(j * 16, 16)] = jnp.zeros((16,), jnp.int32)
        obuf_d[pl.ds(j * 16, 16)] = jnp.full((16,), R, jnp.int32)
        return 0

    lax.fori_loop(0, FB // 16, sent2, 0)
    pltpu.sync_copy(obuf_s.at[pl.ds(0, FB)],
                    ls_hbm.at[pl.ds(pl.multiple_of(w * CAP + fl + FB, 8), FB)])
    pltpu.sync_copy(obuf_d.at[pl.ds(0, FB)],
                    ld_hbm.at[pl.ds(pl.multiple_of(w * CAP + fl + FB, 8), FB)])

    n = fl + p_s
    cvec[pl.ds(0, 16)] = jnp.broadcast_to(n, (16,))
    pltpu.sync_copy(cvec, cnt_hbm.at[pl.ds(pl.multiple_of(w * 16, 8), 16)])

    nseg = (n + SEG - 1) // SEG
    prevsel = jnp.maximum(lane - 1, 0)
    nextsel = jnp.minimum(lane + 1, 15)
    ones = jnp.ones((16,), jnp.int32)

    def rank_of(dl):
        sk, perm = plsc.sort_key_val(dl, lane)
        tmpa[pl.ds(0, 16)] = sk
        prev = plsc.load_gather(tmpa, [prevsel])
        newg = jnp.where(lane == 0, ones, (sk != prev).astype(jnp.int32))
        start = plsc.cummax(jnp.where(newg == 1, lane, 0))
        rank = lane - start
        tmpb[pl.ds(0, 16)] = newg
        nxt = plsc.load_gather(tmpb, [nextsel])
        islast = (lane == 15) | (nxt == 1)
        return sk, perm, rank, islast

    def seg_body(s, _):
        sb = pl.multiple_of(w * CAP + s * SEG, 8)
        pltpu.sync_copy(ls_hbm.at[pl.ds(sb, SEG)], obuf_s.at[pl.ds(0, SEG)])
        pltpu.sync_copy(ld_hbm.at[pl.ds(sb, SEG)], obuf_d.at[pl.ds(0, SEG)])

        for j in range(NB // 16):
            hist[pl.ds(j * 16, 16)] = jnp.zeros((16,), jnp.int32)

        def hvec(v, _):
            dl = obuf_d[pl.ds(v * 16, 16)]
            sk, _, rank, islast = rank_of(dl)
            plsc.addupdate_scatter(hist, [sk], rank + 1, mask=islast)
            return 0

        lax.fori_loop(0, SEG // 16, hvec, 0)

        pltpu.sync_copy(
            hist,
            cnt2_hbm.at[pl.ds(pl.multiple_of((w * MAXSEG + s) * NB, 8), NB)])

        run = jnp.int32(0)
        for j in range(NB // 16):
            h = hist[pl.ds(j * 16, 16)]
            cs = plsc.cumsum(h)
            pos[pl.ds(j * 16, 16)] = run + cs - h
            run = run + jnp.max(cs)

        def svec(v, _):
            dl = obuf_d[pl.ds(v * 16, 16)]
            sv = obuf_s[pl.ds(v * 16, 16)]
            sk, perm, rank, islast = rank_of(dl)
            base = plsc.load_gather(pos, [sk])
            mypos = base + rank
            tmpa[pl.ds(0, 16)] = sv
            svals = plsc.load_gather(tmpa, [perm])
            plsc.store_scatter(sbuf, [mypos], svals)
            plsc.store_scatter(pos, [sk], mypos + 1, mask=islast)
            return 0

        lax.fori_loop(0, SEG // 16, svec, 0)
        pltpu.sync_copy(sbuf.at[pl.ds(0, SEG)], ls_hbm.at[pl.ds(sb, SEG)])
        return 0

    lax.fori_loop(0, nseg, seg_body, 0)


@jax.jit
def _bin_edges(src, dst):
    f = pl.kernel(
        _bin_body,
        out_type=(
            jax.ShapeDtypeStruct((NW * CAP,), jnp.int32),
            jax.ShapeDtypeStruct((NW * CAP,), jnp.int32),
            jax.ShapeDtypeStruct((NW * 16,), jnp.int32),
            jax.ShapeDtypeStruct((NW * MAXSEG * NB,), jnp.int32),
        ),
        mesh=_mesh(),
        compiler_params=pltpu.CompilerParams(needs_layout_passes=False),
        scratch_types=[
            pltpu.VMEM((2 * CHUNK,), jnp.int32),
            pltpu.VMEM((2 * CHUNK,), jnp.int32),
            pltpu.VMEM((OB,), jnp.int32),
            pltpu.VMEM((OB,), jnp.int32),
            pltpu.VMEM((NB,), jnp.int32),
            pltpu.VMEM((NB,), jnp.int32),
            pltpu.VMEM((16,), jnp.int32),
            pltpu.VMEM((16,), jnp.int32),
            pltpu.VMEM((16,), jnp.int32),
            pltpu.SemaphoreType.DMA((2, 2)),
        ],
    )
    return f(src, dst)


def _segmax_body(hp_hbm, ls_hbm, cnt_hbm, cnt2_hbm, out_hbm,
                 agg, rows0, rows1, rows2, rows3,
                 six0, six1, six2, six3, cnts, cvec, gsem, isem):
    w = _wid()
    rows = [rows0, rows1, rows2, rows3]
    six = [six0, six1, six2, six3]

    ninf = jnp.full((16,), -8323200, jnp.int32)

    def init(r, _):
        for c in range(D // 32):
            agg[r, pl.ds(c * 16, 16)] = ninf
        return 0

    lax.fori_loop(0, R + 1, init, 0)

    pltpu.sync_copy(cnt_hbm.at[pl.ds(pl.multiple_of(w * 16, 8), 16)], cvec)
    n = jnp.max(cvec[pl.ds(0, 16)])
    nch = (n + C - 1) // C
    last = jnp.maximum(nch - 1, 0)

    pltpu.sync_copy(
        cnt2_hbm.at[pl.ds(pl.multiple_of(w * MAXSEG * NB, 8), MAXSEG * NB)],
        cnts)

    def cc(k):
        return jnp.minimum(k, last)

    def idx_copy(ch, slot):
        base = pl.multiple_of(w * CAP + ch * C, 8)
        return pltpu.make_async_copy(ls_hbm.at[pl.ds(base, C)], six[slot], isem)

    def gather(slot):
        return pltpu.make_async_copy(hp_hbm.at[six[slot]], rows[slot], gsem)

    def cnt_at(idx):
        v = plsc.load_gather(
            cnts, [jnp.broadcast_to(idx, (16,)).astype(jnp.int32)])
        return jnp.max(v)

    def process(g, slot, state):
        rbuf = rows[slot]

        def do(state):
            r, rem = state
            sgb = (g // SEGC) * NB
            r, rem = lax.cond(
                g % SEGC == 0,
                lambda s: (jnp.int32(0), cnt_at(sgb)),
                lambda s: s, (r, rem))
            left = jnp.minimum(C, n - g * C)

            def outer_cond(st):
                return st[0] < left

            def outer(st):
                e, r, rem = st

                def adv_cond(st2):
                    r2, rem2 = st2
                    return (rem2 == 0) & (r2 < NB - 1)

                def adv(st2):
                    r2, _ = st2
                    return r2 + 1, cnt_at(sgb + r2 + 1)

                r, rem = lax.while_loop(adv_cond, adv, (r, rem))
                k = jnp.minimum(rem, left - e)
                acc = tuple(
                    plsc.bitcast(agg[r, pl.ds(c * 16, 16)], jnp.bfloat16)
                    for c in range(D // 32))

                def red(i, acc):
                    return tuple(
                        jnp.maximum(acc[c], plsc.bitcast(
                            rbuf[e + i, pl.ds(c * 16, 16)], jnp.bfloat16))
                        for c in range(D // 32))

                acc = lax.fori_loop(0, k, red, acc)
                for c in range(D // 32):
                    agg[r, pl.ds(c * 16, 16)] = plsc.bitcast(acc[c], jnp.int32)
                return e + k, r, rem - k

            e, r, rem = lax.while_loop(outer_cond, outer,
                                       (jnp.int32(0), r, rem))
            return r, rem

        return lax.cond(g < nch, do, lambda s: s, state)

    for b in range(4):
        idx_copy(cc(b), b).start()
    for b in range(3):
        idx_copy(cc(b), b).wait()
        gather(b).start()

    ng4 = (nch + 3) // 4

    def quad(g4, state):
        gq = g4 * 4
        for b in range(4):
            g = gq + b
            idx_copy(cc(g + 3), (b + 3) % 4).wait()
            gather((b + 3) % 4).start()
            gather(b).wait()
            idx_copy(cc(g + 4), b).start()
            state = process(g, b, state)
        return state

    lax.fori_loop(0, jnp.maximum(ng4, 1), quad,
                  (jnp.int32(0), jnp.int32(0)))

    for b in range(3):
        gather(b).wait()
    idx_copy(cc(0), 0).wait()

    pltpu.sync_copy(agg.at[pl.ds(0, R)],
                    out_hbm.at[pl.ds(pl.multiple_of(w * R, 8), R)])


@jax.jit
def _segmax(hp, ls, cnt, cnt2):
    f = pl.kernel(
        _segmax_body,
        out_type=jax.ShapeDtypeStruct((NP, D // 2), jnp.int32),
        mesh=_mesh(),
        compiler_params=pltpu.CompilerParams(
            needs_layout_passes=False, use_tc_tiling_on_sc=False),
        scratch_types=(
            [pltpu.VMEM((R + 1, D // 2), jnp.int32)]
            + [pltpu.VMEM((C, D // 2), jnp.int32) for _ in range(4)]
            + [pltpu.VMEM((C,), jnp.int32) for _ in range(4)]
            + [pltpu.VMEM((MAXSEG * NB,), jnp.int32),
               pltpu.VMEM((16,), jnp.int32),
               pltpu.SemaphoreType.DMA,
               pltpu.SemaphoreType.DMA]
        ),
    )
    return _unpack(f(_pack(hp), ls, cnt, cnt2)[:N])


BM = 1000


def _l2norm(h):
    return h / jnp.maximum(
        jnp.sqrt(jnp.sum(h * h, axis=-1, keepdims=True)), 1e-12)


def _pack(x):
    return lax.bitcast_convert_type(
        x.reshape(x.shape[0], x.shape[1] // 2, 2), jnp.int32)


def _unpack(p):
    b = lax.bitcast_convert_type(p, jnp.bfloat16)
    return b.reshape(p.shape[0], p.shape[1] * 2)


def _pool_body(h_ref, w_ref, b_ref, o_ref):
    o_ref[...] = jnp.maximum(
        jnp.dot(h_ref[...], w_ref[...], preferred_element_type=jnp.float32)
        + b_ref[...], 0.0).astype(jnp.bfloat16)


@jax.jit
def _pool_mm(h, Wp, bp):
    return pl.pallas_call(
        _pool_body,
        grid=(N // BM,),
        in_specs=[
            pl.BlockSpec((BM, D), lambda i: (i, 0)),
            pl.BlockSpec((D, D), lambda i: (0, 0)),
            pl.BlockSpec((1, D), lambda i: (0, 0)),
        ],
        out_specs=pl.BlockSpec((BM, D), lambda i: (i, 0)),
        out_shape=jax.ShapeDtypeStruct((N, D), jnp.bfloat16),
    )(h, Wp, bp.reshape(1, D))


def _combine_body(h_ref, a_ref, ws_ref, wn_ref, b_ref, wp_ref, bp_ref,
                  h1_ref, hp1_ref):
    a = a_ref[...].astype(jnp.float32)
    a = jnp.where(jnp.isfinite(a), a, 0.0)
    r = (jnp.dot(h_ref[...], ws_ref[...], preferred_element_type=jnp.float32)
         + jnp.dot(a, wn_ref[...], preferred_element_type=jnp.float32)
         + b_ref[...])
    h1 = _l2norm(jnp.maximum(r, 0.0))
    h1_ref[...] = h1
    hp1_ref[...] = jnp.maximum(
        jnp.dot(h1, wp_ref[...], preferred_element_type=jnp.float32)
        + bp_ref[...], 0.0).astype(jnp.bfloat16)


@jax.jit
def _combine_pool(h, agg, Ws, Wn, b, Wp, bp):
    return pl.pallas_call(
        _combine_body,
        grid=(N // BM,),
        in_specs=[
            pl.BlockSpec((BM, D), lambda i: (i, 0)),
            pl.BlockSpec((BM, D), lambda i: (i, 0)),
            pl.BlockSpec((D, D), lambda i: (0, 0)),
            pl.BlockSpec((D, D), lambda i: (0, 0)),
            pl.BlockSpec((1, D), lambda i: (0, 0)),
            pl.BlockSpec((D, D), lambda i: (0, 0)),
            pl.BlockSpec((1, D), lambda i: (0, 0)),
        ],
        out_specs=[
            pl.BlockSpec((BM, D), lambda i: (i, 0)),
            pl.BlockSpec((BM, D), lambda i: (i, 0)),
        ],
        out_shape=[
            jax.ShapeDtypeStruct((N, D), jnp.float32),
            jax.ShapeDtypeStruct((N, D), jnp.bfloat16),
        ],
    )(h, agg, Ws, Wn, b.reshape(1, D), Wp, bp.reshape(1, D))


def _final_body(h_ref, a_ref, ws_ref, wn_ref, b_ref, o_ref):
    a = a_ref[...].astype(jnp.float32)
    a = jnp.where(jnp.isfinite(a), a, 0.0)
    r = (jnp.dot(h_ref[...], ws_ref[...], preferred_element_type=jnp.float32)
         + jnp.dot(a, wn_ref[...], preferred_element_type=jnp.float32)
         + b_ref[...])
    m = jnp.max(r, axis=-1, keepdims=True)
    ls = r - m - jnp.log(jnp.sum(jnp.exp(r - m), axis=-1, keepdims=True))
    o_ref[...] = _l2norm(ls)


@jax.jit
def _final(h, agg, Ws, Wn, b):
    do = Ws.shape[1]
    return pl.pallas_call(
        _final_body,
        grid=(N // BM,),
        in_specs=[
            pl.BlockSpec((BM, D), lambda i: (i, 0)),
            pl.BlockSpec((BM, D), lambda i: (i, 0)),
            pl.BlockSpec((D, do), lambda i: (0, 0)),
            pl.BlockSpec((D, do), lambda i: (0, 0)),
            pl.BlockSpec((1, do), lambda i: (0, 0)),
        ],
        out_specs=pl.BlockSpec((BM, do), lambda i: (i, 0)),
        out_shape=jax.ShapeDtypeStruct((N, do), jnp.float32),
    )(h, agg, Ws, Wn, b.reshape(1, do))


def kernel(x, edge_index, Wp0, bp0, Wn0, Ws0, b0,
           Wp1, bp1, Wn1, Ws1, b1, Wp2, bp2, Wn2, Ws2, b2):
    src = edge_index[0]
    dst = edge_index[1]
    ls, ld, cnt, cnt2 = _bin_edges(src, dst)
    hp0 = _pool_mm(x, Wp0, bp0)
    agg0 = _segmax(hp0, ls, cnt, cnt2)
    h1, hp1 = _combine_pool(x, agg0, Ws0, Wn0, b0, Wp1, bp1)
    agg1 = _segmax(hp1, ls, cnt, cnt2)
    h2, hp2 = _combine_pool(h1, agg1, Ws1, Wn1, b1, Wp2, bp2)
    agg2 = _segmax(hp2, ls, cnt, cnt2)
    return _final(h2, agg2, Ws2, Wn2, b2)
